# baseline (speedup 1.0000x reference)
import functools

import jax
import jax.numpy as jnp
from jax import lax
from jax.experimental import pallas as pl
from jax.experimental.pallas import tpu as pltpu
from jax.experimental.pallas import tpu_sc as plsc

N = 10000
E = 320000
ND = 128
ED = 16
HID = 128
L = 4

NC, NS = 2, 16
NW = NC * NS
EPW = E // NW
CHUNK = 80
NCHUNKS = EPW // CHUNK
NACC = 10240
ROWS_PT = NACC // NS

_SC_MESH = plsc.VectorSubcoreMesh(core_axis_name="c", subcore_axis_name="s",
                                  num_cores=NC, num_subcores=NS)


def _proj_body(hn_ref, ws_ref, wd_ref, ps_ref, pd_ref):
    hn = hn_ref[...]
    ps_ref[...] = jnp.dot(hn, ws_ref[...], preferred_element_type=jnp.float32, precision=lax.Precision.HIGHEST)
    pd_ref[...] = jnp.dot(hn, wd_ref[...], preferred_element_type=jnp.float32, precision=lax.Precision.HIGHEST)


def _proj(h_node, w1s, w1d):
    bn = 2000
    return pl.pallas_call(
        _proj_body,
        grid=(N // bn,),
        in_specs=[
            pl.BlockSpec((bn, ND), lambda i: (i, 0)),
            pl.BlockSpec((ND, HID), lambda i: (0, 0)),
            pl.BlockSpec((ND, HID), lambda i: (0, 0)),
        ],
        out_specs=[
            pl.BlockSpec((bn, HID), lambda i: (i, 0)),
            pl.BlockSpec((bn, HID), lambda i: (i, 0)),
        ],
        out_shape=[
            jax.ShapeDtypeStruct((N, HID), jnp.float32),
            jax.ShapeDtypeStruct((N, HID), jnp.float32),
        ],
    )(h_node, w1s, w1d)


def _edge_body(g_ref, he_ref, w1e_ref, b1_ref, w2_ref, b2_ref,
               lg_ref, lb_ref, out_ref):
    he = he_ref[...]
    pre = (g_ref[...]
           + jnp.dot(he, w1e_ref[...], preferred_element_type=jnp.float32, precision=lax.Precision.HIGHEST)
           + b1_ref[...])
    h = jnp.maximum(pre, 0.0)
    e = jnp.dot(h, w2_ref[...], preferred_element_type=jnp.float32, precision=lax.Precision.HIGHEST) + b2_ref[...]
    m = jnp.mean(e, axis=-1, keepdims=True)
    v = jnp.mean((e - m) ** 2, axis=-1, keepdims=True)
    e = (e - m) * lax.rsqrt(v + 1e-5) * lg_ref[...] + lb_ref[...]
    out_ref[...] = he + e


def _edge_mlp(g, h_edge, w1e, b1, w2, b2, ln_g, ln_b):
    e_len = g.shape[0]
    be = 2000 if e_len % 2560 else 2560
    return pl.pallas_call(
        _edge_body,
        grid=(e_len // be,),
        in_specs=[
            pl.BlockSpec((be, HID), lambda i: (i, 0)),
            pl.BlockSpec((be, ED), lambda i: (i, 0)),
            pl.BlockSpec((ED, HID), lambda i: (0, 0)),
            pl.BlockSpec((1, HID), lambda i: (0, 0)),
            pl.BlockSpec((HID, ED), lambda i: (0, 0)),
            pl.BlockSpec((1, ED), lambda i: (0, 0)),
            pl.BlockSpec((1, ED), lambda i: (0, 0)),
            pl.BlockSpec((1, ED), lambda i: (0, 0)),
        ],
        out_specs=pl.BlockSpec((be, ED), lambda i: (i, 0)),
        out_shape=jax.ShapeDtypeStruct((e_len, ED), jnp.float32),
    )(g, h_edge, w1e, b1.reshape(1, HID), w2, b2.reshape(1, ED),
      ln_g.reshape(1, ED), ln_b.reshape(1, ED))


def _node_body(hn_ref, a0_ref, a1_ref, w1n_ref, w1a_ref, b1_ref, w2_ref,
               b2_ref, lg_ref, lb_ref, ws_ref, wd_ref,
               out_ref, ps_ref, pd_ref):
    hn = hn_ref[...]
    agg = a0_ref[0, :, :ED] + a1_ref[0, :, :ED]
    pre = (jnp.dot(hn, w1n_ref[...], preferred_element_type=jnp.float32, precision=lax.Precision.HIGHEST)
           + jnp.dot(agg, w1a_ref[...], preferred_element_type=jnp.float32, precision=lax.Precision.HIGHEST)
           + b1_ref[...])
    h = jnp.maximum(pre, 0.0)
    nv = jnp.dot(h, w2_ref[...], preferred_element_type=jnp.float32, precision=lax.Precision.HIGHEST) + b2_ref[...]
    m = jnp.mean(nv, axis=-1, keepdims=True)
    v = jnp.mean((nv - m) ** 2, axis=-1, keepdims=True)
    nv = (nv - m) * lax.rsqrt(v + 1e-5) * lg_ref[...] + lb_ref[...]
    hn_new = hn + nv
    out_ref[...] = hn_new
    ps_ref[...] = jnp.dot(hn_new, ws_ref[...], preferred_element_type=jnp.float32, precision=lax.Precision.HIGHEST)
    pd_ref[...] = jnp.dot(hn_new, wd_ref[...], preferred_element_type=jnp.float32, precision=lax.Precision.HIGHEST)


def _node_mlp(h_node, parts, w1, b1, w2, b2, ln_g, ln_b, w1s_next, w1d_next):
    bn = 2000
    return pl.pallas_call(
        _node_body,
        grid=(N // bn,),
        in_specs=[
            pl.BlockSpec((bn, ND), lambda i: (i, 0)),
            pl.BlockSpec((1, bn, ND), lambda i: (0, i, 0)),
            pl.BlockSpec((1, bn, ND), lambda i: (1, i, 0)),
            pl.BlockSpec((ND, HID), lambda i: (0, 0)),
            pl.BlockSpec((ED, HID), lambda i: (0, 0)),
            pl.BlockSpec((1, HID), lambda i: (0, 0)),
            pl.BlockSpec((HID, ND), lambda i: (0, 0)),
            pl.BlockSpec((1, ND), lambda i: (0, 0)),
            pl.BlockSpec((1, ND), lambda i: (0, 0)),
            pl.BlockSpec((1, ND), lambda i: (0, 0)),
            pl.BlockSpec((ND, HID), lambda i: (0, 0)),
            pl.BlockSpec((ND, HID), lambda i: (0, 0)),
        ],
        out_specs=[
            pl.BlockSpec((bn, ND), lambda i: (i, 0)),
            pl.BlockSpec((bn, HID), lambda i: (i, 0)),
            pl.BlockSpec((bn, HID), lambda i: (i, 0)),
        ],
        out_shape=[
            jax.ShapeDtypeStruct((N, ND), jnp.float32),
            jax.ShapeDtypeStruct((N, HID), jnp.float32),
            jax.ShapeDtypeStruct((N, HID), jnp.float32),
        ],
    )(h_node, parts, parts, w1[:ND], w1[ND:], b1.reshape(1, HID), w2,
      b2.reshape(1, ND), ln_g.reshape(1, ND), ln_b.reshape(1, ND),
      w1s_next, w1d_next)


def _make_gather(e_total, chunk):
    epw = e_total // NW
    nchunks = epw // chunk
    assert epw % chunk == 0 and chunk % 8 == 0 and chunk <= 128
    assert nchunks % 2 == 1

    scratch = [
        pltpu.VMEM((nchunks, chunk), jnp.int32),
        pltpu.VMEM((nchunks, chunk), jnp.int32),
        pltpu.VMEM((chunk, HID), jnp.float32),
        pltpu.VMEM((chunk, HID), jnp.float32),
        pltpu.VMEM((chunk, HID), jnp.float32),
        pltpu.VMEM((chunk, HID), jnp.float32),
        pltpu.VMEM((chunk, HID), jnp.float32),
        pltpu.VMEM((chunk, HID), jnp.float32),
        pltpu.SemaphoreType.DMA,
        pltpu.SemaphoreType.DMA,
        pltpu.SemaphoreType.DMA,
        pltpu.SemaphoreType.DMA,
    ]

    def body(ps_hbm, pd_hbm, src_hbm, dst_hbm, out_hbm,
             si_all, di_all, rs0, rs1, rd0, rd1, ou0, ou1,
             gsem0, gsem1, wsem0, wsem1):
        wid = lax.axis_index("s") * NC + lax.axis_index("c")
        base = wid * epw
        rs = (rs0, rs1)
        rd = (rd0, rd1)
        ou = (ou0, ou1)
        gsem = (gsem0, gsem1)
        wsem = (wsem0, wsem1)

        pltpu.sync_copy(src_hbm.at[wid], si_all)
        pltpu.sync_copy(dst_hbm.at[wid], di_all)

        def issue(c, p):
            pltpu.async_copy(ps_hbm.at[si_all.at[c]], rs[p], gsem[p])
            pltpu.async_copy(pd_hbm.at[di_all.at[c]], rd[p], gsem[p])

        def process(c, p):
            pltpu.make_async_copy(
                ps_hbm.at[si_all.at[0]], rs[p], gsem[p]).wait()
            pltpu.make_async_copy(
                pd_hbm.at[di_all.at[0]], rd[p], gsem[p]).wait()

            @pl.when(c >= 2)
            def _():
                pltpu.make_async_copy(
                    ou[p], out_hbm.at[pl.ds(0, chunk)], wsem[p]).wait()

            def add_row(r, carry):
                for cix in range(HID // 16):
                    sl = pl.ds(cix * 16, 16)
                    ou[p][r, sl] = rs[p][r, sl] + rd[p][r, sl]
                return carry

            lax.fori_loop(0, chunk, add_row, 0)
            pltpu.async_copy(
                ou[p], out_hbm.at[pl.ds(base + c * chunk, chunk)], wsem[p])

        issue(0, 0)

        def loop_body(j2, carry):
            c0 = 2 * j2
            issue(c0 + 1, 1)
            process(c0, 0)
            issue(c0 + 2, 0)
            process(c0 + 1, 1)
            return carry

        lax.fori_loop(0, (nchunks - 1) // 2, loop_body, 0)
        process(nchunks - 1, 0)
        pltpu.make_async_copy(
            ou[0], out_hbm.at[pl.ds(0, chunk)], wsem[0]).wait()
        pltpu.make_async_copy(
            ou[1], out_hbm.at[pl.ds(0, chunk)], wsem[1]).wait()

    return pl.kernel(
        body,
        out_type=jax.ShapeDtypeStruct((e_total, HID), jnp.float32),
        mesh=_SC_MESH,
        scratch_types=scratch,
    )


_gather_add = _make_gather(E, CHUNK)
_gather_half = _make_gather(E // 2, 40)

_SEGSUM_SCRATCH = [
    pltpu.VMEM((NCHUNKS, CHUNK), jnp.int32),
    pltpu.VMEM((CHUNK, ED), jnp.float32),
    pltpu.VMEM((CHUNK, ED), jnp.float32),
    pltpu.VMEM((CHUNK, ND), jnp.float32),
    pltpu.VMEM_SHARED((NACC, ND), jnp.float32),
    pltpu.SemaphoreType.DMA,
    pltpu.SemaphoreType.DMA,
    pltpu.SemaphoreType.DMA,
]


def _segment_sum_body(he_hbm, dst_hbm, out_hbm, di_all, r16a, r16b,
                      rows_v, acc_sh, lsem0, lsem1, ssem):
    c = lax.axis_index("c")
    s = lax.axis_index("s")
    wid = s * NC + c
    r16 = (r16a, r16b)
    lsem = (lsem0, lsem1)

    def zrow(r, carry):
        for cix in range(ND // 16):
            rows_v[r, pl.ds(cix * 16, 16)] = jnp.zeros((16,), jnp.float32)
        return carry

    lax.fori_loop(0, CHUNK, zrow, 0)
    for t in range(NACC // NS // CHUNK):
        pltpu.sync_copy(
            rows_v,
            acc_sh.at[pl.ds(s * (NACC // NS) + t * CHUNK, CHUNK)])
    plsc.subcore_barrier()

    pltpu.sync_copy(dst_hbm.at[wid], di_all)

    def issue(cc, p):
        pltpu.async_copy(he_hbm.at[wid, pl.ds(cc * CHUNK, CHUNK)],
                         r16[p], lsem[p])

    def process(cc, p):
        pltpu.make_async_copy(
            he_hbm.at[wid, pl.ds(0, CHUNK)], r16[p], lsem[p]).wait()

        @pl.when(cc >= 1)
        def _():
            pltpu.make_async_copy(
                rows_v, acc_sh.at[di_all.at[0]], ssem).wait()

        def fill(r, carry):
            rows_v[r, pl.ds(0, 16)] = r16[p][r, pl.ds(0, 16)]
            return carry

        lax.fori_loop(0, CHUNK, fill, 0)
        pltpu.async_copy(rows_v, acc_sh.at[di_all.at[cc]], ssem,
                         add=True)

    issue(0, 0)

    def loop_body(j2, carry):
        c0 = 2 * j2
        issue(c0 + 1, 1)
        process(c0, 0)
        issue(c0 + 2, 0)
        process(c0 + 1, 1)
        return carry

    lax.fori_loop(0, (NCHUNKS - 1) // 2, loop_body, 0)
    process(NCHUNKS - 1, 0)
    pltpu.make_async_copy(rows_v, acc_sh.at[di_all.at[0]], ssem).wait()
    plsc.subcore_barrier()

    @pl.when(s == 0)
    def _():
        pltpu.sync_copy(acc_sh, out_hbm.at[c])


_segment_sum = pl.kernel(
    _segment_sum_body,
    out_type=jax.ShapeDtypeStruct((NC, NACC, ND), jnp.float32),
    mesh=_SC_MESH,
    scratch_types=_SEGSUM_SCRATCH,
)


def kernel(h_node, edge_index, h_edge, edge_w1, edge_b1, edge_w2, edge_b2,
           edge_ln_g, edge_ln_b, node_w1, node_b1, node_w2, node_b2,
           node_ln_g, node_ln_b):
    E2 = E // 2
    src = edge_index[0]
    dst = edge_index[1]
    dst4 = dst.reshape(NW, NCHUNKS, CHUNK)
    idx_halves = []
    for h in range(2):
        sl = slice(h * E2, (h + 1) * E2)
        idx_halves.append((src[sl].reshape(NW, -1, 40),
                           dst[sl].reshape(NW, -1, 40)))
    ps, pd = _proj(h_node, edge_w1[0, :ND], edge_w1[0, ND:2 * ND])
    for i in range(L):
        nxt = (i + 1) % L
        he_halves = []
        for h in range(2):
            sh, dh = idx_halves[h]
            g = _gather_half(ps, pd, sh, dh)
            he_halves.append(
                _edge_mlp(g, lax.slice_in_dim(h_edge, h * E2, (h + 1) * E2),
                          edge_w1[i, 2 * ND:], edge_b1[i], edge_w2[i],
                          edge_b2[i], edge_ln_g[i], edge_ln_b[i]))
        h_edge = jnp.concatenate(he_halves, axis=0)
        parts = _segment_sum(h_edge.reshape(NW, EPW, ED), dst4)
        h_node, ps, pd = _node_mlp(h_node, parts, node_w1[i], node_b1[i],
                                   node_w2[i], node_b2[i], node_ln_g[i],
                                   node_ln_b[i], edge_w1[nxt, :ND],
                                   edge_w1[nxt, ND:2 * ND])
    return (h_node, h_edge)


# speedup vs baseline: 1.0556x; 1.0556x over previous
import functools

import jax
import jax.numpy as jnp
from jax import lax
from jax.experimental import pallas as pl
from jax.experimental.pallas import tpu as pltpu
from jax.experimental.pallas import tpu_sc as plsc

N = 10000
E = 320000
ND = 128
ED = 16
HID = 128
L = 4

NC, NS = 2, 16
NW = NC * NS
EPW = E // NW
CHUNK = 80
NCHUNKS = EPW // CHUNK
NACC = 10240
ROWS_PT = NACC // NS

_SC_MESH = plsc.VectorSubcoreMesh(core_axis_name="c", subcore_axis_name="s",
                                  num_cores=NC, num_subcores=NS)


def _proj_body(hn_ref, ws_ref, wd_ref, ps_ref, pd_ref):
    hn = hn_ref[...]
    ps_ref[...] = jnp.dot(hn, ws_ref[...], preferred_element_type=jnp.float32, precision=lax.Precision.HIGHEST)
    pd_ref[...] = jnp.dot(hn, wd_ref[...], preferred_element_type=jnp.float32, precision=lax.Precision.HIGHEST)


def _proj(h_node, w1s, w1d):
    bn = 2000
    return pl.pallas_call(
        _proj_body,
        grid=(N // bn,),
        in_specs=[
            pl.BlockSpec((bn, ND), lambda i: (i, 0)),
            pl.BlockSpec((ND, HID), lambda i: (0, 0)),
            pl.BlockSpec((ND, HID), lambda i: (0, 0)),
        ],
        out_specs=[
            pl.BlockSpec((bn, HID), lambda i: (i, 0)),
            pl.BlockSpec((bn, HID), lambda i: (i, 0)),
        ],
        out_shape=[
            jax.ShapeDtypeStruct((N, HID), jnp.float32),
            jax.ShapeDtypeStruct((N, HID), jnp.float32),
        ],
    )(h_node, w1s, w1d)


def _edge_body(g_ref, he_ref, w1e_ref, b1_ref, w2_ref, b2_ref,
               lg_ref, lb_ref, out_ref):
    he = he_ref[...]
    pre = (g_ref[...]
           + jnp.dot(he, w1e_ref[...], preferred_element_type=jnp.float32, precision=lax.Precision.HIGHEST)
           + b1_ref[...])
    h = jnp.maximum(pre, 0.0)
    e = jnp.dot(h, w2_ref[...], preferred_element_type=jnp.float32, precision=lax.Precision.HIGHEST) + b2_ref[...]
    m = jnp.mean(e, axis=-1, keepdims=True)
    v = jnp.mean((e - m) ** 2, axis=-1, keepdims=True)
    e = (e - m) * lax.rsqrt(v + 1e-5) * lg_ref[...] + lb_ref[...]
    out_ref[...] = he + e


def _edge_mlp(g, h_edge, w1e, b1, w2, b2, ln_g, ln_b):
    e_len = g.shape[0]
    be = 2000 if e_len % 2560 else 2560
    return pl.pallas_call(
        _edge_body,
        grid=(e_len // be,),
        in_specs=[
            pl.BlockSpec((be, HID), lambda i: (i, 0)),
            pl.BlockSpec((be, ED), lambda i: (i, 0)),
            pl.BlockSpec((ED, HID), lambda i: (0, 0)),
            pl.BlockSpec((1, HID), lambda i: (0, 0)),
            pl.BlockSpec((HID, ED), lambda i: (0, 0)),
            pl.BlockSpec((1, ED), lambda i: (0, 0)),
            pl.BlockSpec((1, ED), lambda i: (0, 0)),
            pl.BlockSpec((1, ED), lambda i: (0, 0)),
        ],
        out_specs=pl.BlockSpec((be, ED), lambda i: (i, 0)),
        out_shape=jax.ShapeDtypeStruct((e_len, ED), jnp.float32),
    )(g, h_edge, w1e, b1.reshape(1, HID), w2, b2.reshape(1, ED),
      ln_g.reshape(1, ED), ln_b.reshape(1, ED))


def _node_body(hn_ref, a0_ref, a1_ref, w1n_ref, w1a_ref, b1_ref, w2_ref,
               b2_ref, lg_ref, lb_ref, ws_ref, wd_ref,
               out_ref, ps_ref, pd_ref):
    hn = hn_ref[...]
    agg = a0_ref[0, :, :ED] + a1_ref[0, :, :ED]
    pre = (jnp.dot(hn, w1n_ref[...], preferred_element_type=jnp.float32, precision=lax.Precision.HIGHEST)
           + jnp.dot(agg, w1a_ref[...], preferred_element_type=jnp.float32, precision=lax.Precision.HIGHEST)
           + b1_ref[...])
    h = jnp.maximum(pre, 0.0)
    nv = jnp.dot(h, w2_ref[...], preferred_element_type=jnp.float32, precision=lax.Precision.HIGHEST) + b2_ref[...]
    m = jnp.mean(nv, axis=-1, keepdims=True)
    v = jnp.mean((nv - m) ** 2, axis=-1, keepdims=True)
    nv = (nv - m) * lax.rsqrt(v + 1e-5) * lg_ref[...] + lb_ref[...]
    hn_new = hn + nv
    out_ref[...] = hn_new
    ps_ref[...] = jnp.dot(hn_new, ws_ref[...], preferred_element_type=jnp.float32, precision=lax.Precision.HIGHEST)
    pd_ref[...] = jnp.dot(hn_new, wd_ref[...], preferred_element_type=jnp.float32, precision=lax.Precision.HIGHEST)


def _node_mlp(h_node, parts, w1, b1, w2, b2, ln_g, ln_b, w1s_next, w1d_next):
    bn = 2000
    return pl.pallas_call(
        _node_body,
        grid=(N // bn,),
        in_specs=[
            pl.BlockSpec((bn, ND), lambda i: (i, 0)),
            pl.BlockSpec((1, bn, ND), lambda i: (0, i, 0)),
            pl.BlockSpec((1, bn, ND), lambda i: (1, i, 0)),
            pl.BlockSpec((ND, HID), lambda i: (0, 0)),
            pl.BlockSpec((ED, HID), lambda i: (0, 0)),
            pl.BlockSpec((1, HID), lambda i: (0, 0)),
            pl.BlockSpec((HID, ND), lambda i: (0, 0)),
            pl.BlockSpec((1, ND), lambda i: (0, 0)),
            pl.BlockSpec((1, ND), lambda i: (0, 0)),
            pl.BlockSpec((1, ND), lambda i: (0, 0)),
            pl.BlockSpec((ND, HID), lambda i: (0, 0)),
            pl.BlockSpec((ND, HID), lambda i: (0, 0)),
        ],
        out_specs=[
            pl.BlockSpec((bn, ND), lambda i: (i, 0)),
            pl.BlockSpec((bn, HID), lambda i: (i, 0)),
            pl.BlockSpec((bn, HID), lambda i: (i, 0)),
        ],
        out_shape=[
            jax.ShapeDtypeStruct((N, ND), jnp.float32),
            jax.ShapeDtypeStruct((N, HID), jnp.float32),
            jax.ShapeDtypeStruct((N, HID), jnp.float32),
        ],
    )(h_node, parts, parts, w1[:ND], w1[ND:], b1.reshape(1, HID), w2,
      b2.reshape(1, ND), ln_g.reshape(1, ND), ln_b.reshape(1, ND),
      w1s_next, w1d_next)


def _make_gather(e_total, chunk):
    epw = e_total // NW
    nchunks = epw // chunk
    assert epw % chunk == 0 and chunk % 8 == 0 and chunk <= 128
    assert nchunks % 2 == 1

    scratch = [
        pltpu.VMEM((nchunks, chunk), jnp.int32),
        pltpu.VMEM((nchunks, chunk), jnp.int32),
        pltpu.VMEM((chunk, HID), jnp.float32),
        pltpu.VMEM((chunk, HID), jnp.float32),
        pltpu.VMEM((chunk, HID), jnp.float32),
        pltpu.VMEM((chunk, HID), jnp.float32),
        pltpu.VMEM((chunk, HID), jnp.float32),
        pltpu.VMEM((chunk, HID), jnp.float32),
        pltpu.SemaphoreType.DMA,
        pltpu.SemaphoreType.DMA,
        pltpu.SemaphoreType.DMA,
        pltpu.SemaphoreType.DMA,
    ]

    def body(ps_hbm, pd_hbm, src_hbm, dst_hbm, out_hbm,
             si_all, di_all, rs0, rs1, rd0, rd1, ou0, ou1,
             gsem0, gsem1, wsem0, wsem1):
        wid = lax.axis_index("s") * NC + lax.axis_index("c")
        base = wid * epw
        rs = (rs0, rs1)
        rd = (rd0, rd1)
        ou = (ou0, ou1)
        gsem = (gsem0, gsem1)
        wsem = (wsem0, wsem1)

        pltpu.sync_copy(src_hbm.at[wid], si_all)
        pltpu.sync_copy(dst_hbm.at[wid], di_all)

        def issue(c, p):
            pltpu.async_copy(ps_hbm.at[si_all.at[c]], rs[p], gsem[p])
            pltpu.async_copy(pd_hbm.at[di_all.at[c]], rd[p], gsem[p])

        def process(c, p):
            pltpu.make_async_copy(
                ps_hbm.at[si_all.at[0]], rs[p], gsem[p]).wait()
            pltpu.make_async_copy(
                pd_hbm.at[di_all.at[0]], rd[p], gsem[p]).wait()

            @pl.when(c >= 2)
            def _():
                pltpu.make_async_copy(
                    ou[p], out_hbm.at[pl.ds(0, chunk)], wsem[p]).wait()

            def add_row(r, carry):
                for cix in range(HID // 16):
                    sl = pl.ds(cix * 16, 16)
                    ou[p][r, sl] = rs[p][r, sl] + rd[p][r, sl]
                return carry

            lax.fori_loop(0, chunk, add_row, 0)
            pltpu.async_copy(
                ou[p], out_hbm.at[pl.ds(base + c * chunk, chunk)], wsem[p])

        issue(0, 0)

        def loop_body(j2, carry):
            c0 = 2 * j2
            issue(c0 + 1, 1)
            process(c0, 0)
            issue(c0 + 2, 0)
            process(c0 + 1, 1)
            return carry

        lax.fori_loop(0, (nchunks - 1) // 2, loop_body, 0)
        process(nchunks - 1, 0)
        pltpu.make_async_copy(
            ou[0], out_hbm.at[pl.ds(0, chunk)], wsem[0]).wait()
        pltpu.make_async_copy(
            ou[1], out_hbm.at[pl.ds(0, chunk)], wsem[1]).wait()

    return pl.kernel(
        body,
        out_type=jax.ShapeDtypeStruct((e_total, HID), jnp.float32),
        mesh=_SC_MESH,
        scratch_types=scratch,
    )


_gather_add = _make_gather(E, CHUNK)

_SEGSUM_SCRATCH = [
    pltpu.VMEM((NCHUNKS, CHUNK), jnp.int32),
    pltpu.VMEM((CHUNK, ED), jnp.float32),
    pltpu.VMEM((CHUNK, ED), jnp.float32),
    pltpu.VMEM((CHUNK, ND), jnp.float32),
    pltpu.VMEM_SHARED((NACC, ND), jnp.float32),
    pltpu.SemaphoreType.DMA,
    pltpu.SemaphoreType.DMA,
    pltpu.SemaphoreType.DMA,
]


def _segment_sum_body(he_hbm, dst_hbm, out_hbm, di_all, r16a, r16b,
                      rows_v, acc_sh, lsem0, lsem1, ssem):
    c = lax.axis_index("c")
    s = lax.axis_index("s")
    wid = s * NC + c
    r16 = (r16a, r16b)
    lsem = (lsem0, lsem1)

    def zrow(r, carry):
        for cix in range(ND // 16):
            rows_v[r, pl.ds(cix * 16, 16)] = jnp.zeros((16,), jnp.float32)
        return carry

    lax.fori_loop(0, CHUNK, zrow, 0)
    for t in range(NACC // NS // CHUNK):
        pltpu.sync_copy(
            rows_v,
            acc_sh.at[pl.ds(s * (NACC // NS) + t * CHUNK, CHUNK)])
    plsc.subcore_barrier()

    pltpu.sync_copy(dst_hbm.at[wid], di_all)

    def issue(cc, p):
        pltpu.async_copy(he_hbm.at[wid, pl.ds(cc * CHUNK, CHUNK)],
                         r16[p], lsem[p])

    def process(cc, p):
        pltpu.make_async_copy(
            he_hbm.at[wid, pl.ds(0, CHUNK)], r16[p], lsem[p]).wait()

        @pl.when(cc >= 1)
        def _():
            pltpu.make_async_copy(
                rows_v, acc_sh.at[di_all.at[0]], ssem).wait()

        def fill(r, carry):
            rows_v[r, pl.ds(0, 16)] = r16[p][r, pl.ds(0, 16)]
            return carry

        lax.fori_loop(0, CHUNK, fill, 0)
        pltpu.async_copy(rows_v, acc_sh.at[di_all.at[cc]], ssem,
                         add=True)

    issue(0, 0)

    def loop_body(j2, carry):
        c0 = 2 * j2
        issue(c0 + 1, 1)
        process(c0, 0)
        issue(c0 + 2, 0)
        process(c0 + 1, 1)
        return carry

    lax.fori_loop(0, (NCHUNKS - 1) // 2, loop_body, 0)
    process(NCHUNKS - 1, 0)
    pltpu.make_async_copy(rows_v, acc_sh.at[di_all.at[0]], ssem).wait()
    plsc.subcore_barrier()

    @pl.when(s == 0)
    def _():
        pltpu.sync_copy(acc_sh, out_hbm.at[c])


_segment_sum = pl.kernel(
    _segment_sum_body,
    out_type=jax.ShapeDtypeStruct((NC, NACC, ND), jnp.float32),
    mesh=_SC_MESH,
    scratch_types=_SEGSUM_SCRATCH,
)


def kernel(h_node, edge_index, h_edge, edge_w1, edge_b1, edge_w2, edge_b2,
           edge_ln_g, edge_ln_b, node_w1, node_b1, node_w2, node_b2,
           node_ln_g, node_ln_b):
    src4 = edge_index[0].reshape(NW, NCHUNKS, CHUNK)
    dst4 = edge_index[1].reshape(NW, NCHUNKS, CHUNK)
    ps, pd = _proj(h_node, edge_w1[0, :ND], edge_w1[0, ND:2 * ND])
    for i in range(L):
        nxt = (i + 1) % L
        g = _gather_add(ps, pd, src4, dst4)
        h_edge = _edge_mlp(g, h_edge, edge_w1[i, 2 * ND:], edge_b1[i],
                           edge_w2[i], edge_b2[i], edge_ln_g[i], edge_ln_b[i])
        parts = _segment_sum(h_edge.reshape(NW, EPW, ED), dst4)
        h_node, ps, pd = _node_mlp(h_node, parts, node_w1[i], node_b1[i],
                                   node_w2[i], node_b2[i], node_ln_g[i],
                                   node_ln_b[i], edge_w1[nxt, :ND],
                                   edge_w1[nxt, ND:2 * ND])
    return (h_node, h_edge)


# speedup vs baseline: 1.0800x; 1.0232x over previous
import functools

import jax
import jax.numpy as jnp
from jax import lax
from jax.experimental import pallas as pl
from jax.experimental.pallas import tpu as pltpu
from jax.experimental.pallas import tpu_sc as plsc

N = 10000
E = 320000
ND = 128
ED = 16
HID = 128
L = 4

NC, NS = 2, 16
NW = NC * NS
EPW = E // NW
CHUNK = 80
NCHUNKS = EPW // CHUNK
NACC = 10240
ROWS_PT = NACC // NS

_SC_MESH = plsc.VectorSubcoreMesh(core_axis_name="c", subcore_axis_name="s",
                                  num_cores=NC, num_subcores=NS)


def _proj_body(hn_ref, ws_ref, wd_ref, ps_ref, pd_ref):
    hn = hn_ref[...]
    ps_ref[...] = jnp.dot(hn, ws_ref[...], preferred_element_type=jnp.float32, precision=lax.Precision.HIGHEST)
    pd_ref[...] = jnp.dot(hn, wd_ref[...], preferred_element_type=jnp.float32, precision=lax.Precision.HIGHEST)


def _proj(h_node, w1s, w1d):
    bn = 2000
    return pl.pallas_call(
        _proj_body,
        grid=(N // bn,),
        in_specs=[
            pl.BlockSpec((bn, ND), lambda i: (i, 0)),
            pl.BlockSpec((ND, HID), lambda i: (0, 0)),
            pl.BlockSpec((ND, HID), lambda i: (0, 0)),
        ],
        out_specs=[
            pl.BlockSpec((bn, HID), lambda i: (i, 0)),
            pl.BlockSpec((bn, HID), lambda i: (i, 0)),
        ],
        out_shape=[
            jax.ShapeDtypeStruct((N, HID), jnp.float32),
            jax.ShapeDtypeStruct((N, HID), jnp.float32),
        ],
    )(h_node, w1s, w1d)


def _edge_body(g_ref, he_ref, w1e_ref, b1_ref, w2_ref, b2_ref,
               lg_ref, lb_ref, out_ref):
    he = he_ref[...]
    pre = (g_ref[...]
           + jnp.dot(he, w1e_ref[...], preferred_element_type=jnp.float32, precision=lax.Precision.HIGHEST)
           + b1_ref[...])
    h = jnp.maximum(pre, 0.0)
    e = jnp.dot(h, w2_ref[...], preferred_element_type=jnp.float32, precision=lax.Precision.HIGHEST) + b2_ref[...]
    m = jnp.mean(e, axis=-1, keepdims=True)
    v = jnp.mean((e - m) ** 2, axis=-1, keepdims=True)
    e = (e - m) * lax.rsqrt(v + 1e-5) * lg_ref[...] + lb_ref[...]
    out_ref[...] = he + e


def _edge_mlp(g, h_edge, w1e, b1, w2, b2, ln_g, ln_b):
    e_len = g.shape[0]
    be = 8000
    return pl.pallas_call(
        _edge_body,
        grid=(e_len // be,),
        in_specs=[
            pl.BlockSpec((be, HID), lambda i: (i, 0)),
            pl.BlockSpec((be, ED), lambda i: (i, 0)),
            pl.BlockSpec((ED, HID), lambda i: (0, 0)),
            pl.BlockSpec((1, HID), lambda i: (0, 0)),
            pl.BlockSpec((HID, ED), lambda i: (0, 0)),
            pl.BlockSpec((1, ED), lambda i: (0, 0)),
            pl.BlockSpec((1, ED), lambda i: (0, 0)),
            pl.BlockSpec((1, ED), lambda i: (0, 0)),
        ],
        out_specs=pl.BlockSpec((be, ED), lambda i: (i, 0)),
        out_shape=jax.ShapeDtypeStruct((e_len, ED), jnp.float32),
    )(g, h_edge, w1e, b1.reshape(1, HID), w2, b2.reshape(1, ED),
      ln_g.reshape(1, ED), ln_b.reshape(1, ED))


def _node_body(hn_ref, a0_ref, a1_ref, w1n_ref, w1a_ref, b1_ref, w2_ref,
               b2_ref, lg_ref, lb_ref, ws_ref, wd_ref,
               out_ref, ps_ref, pd_ref):
    hn = hn_ref[...]
    agg = a0_ref[0, :, :ED] + a1_ref[0, :, :ED]
    pre = (jnp.dot(hn, w1n_ref[...], preferred_element_type=jnp.float32, precision=lax.Precision.HIGHEST)
           + jnp.dot(agg, w1a_ref[...], preferred_element_type=jnp.float32, precision=lax.Precision.HIGHEST)
           + b1_ref[...])
    h = jnp.maximum(pre, 0.0)
    nv = jnp.dot(h, w2_ref[...], preferred_element_type=jnp.float32, precision=lax.Precision.HIGHEST) + b2_ref[...]
    m = jnp.mean(nv, axis=-1, keepdims=True)
    v = jnp.mean((nv - m) ** 2, axis=-1, keepdims=True)
    nv = (nv - m) * lax.rsqrt(v + 1e-5) * lg_ref[...] + lb_ref[...]
    hn_new = hn + nv
    out_ref[...] = hn_new
    ps_ref[...] = jnp.dot(hn_new, ws_ref[...], preferred_element_type=jnp.float32, precision=lax.Precision.HIGHEST)
    pd_ref[...] = jnp.dot(hn_new, wd_ref[...], preferred_element_type=jnp.float32, precision=lax.Precision.HIGHEST)


def _node_mlp(h_node, parts, w1, b1, w2, b2, ln_g, ln_b, w1s_next, w1d_next):
    bn = 2000
    return pl.pallas_call(
        _node_body,
        grid=(N // bn,),
        in_specs=[
            pl.BlockSpec((bn, ND), lambda i: (i, 0)),
            pl.BlockSpec((1, bn, ND), lambda i: (0, i, 0)),
            pl.BlockSpec((1, bn, ND), lambda i: (1, i, 0)),
            pl.BlockSpec((ND, HID), lambda i: (0, 0)),
            pl.BlockSpec((ED, HID), lambda i: (0, 0)),
            pl.BlockSpec((1, HID), lambda i: (0, 0)),
            pl.BlockSpec((HID, ND), lambda i: (0, 0)),
            pl.BlockSpec((1, ND), lambda i: (0, 0)),
            pl.BlockSpec((1, ND), lambda i: (0, 0)),
            pl.BlockSpec((1, ND), lambda i: (0, 0)),
            pl.BlockSpec((ND, HID), lambda i: (0, 0)),
            pl.BlockSpec((ND, HID), lambda i: (0, 0)),
        ],
        out_specs=[
            pl.BlockSpec((bn, ND), lambda i: (i, 0)),
            pl.BlockSpec((bn, HID), lambda i: (i, 0)),
            pl.BlockSpec((bn, HID), lambda i: (i, 0)),
        ],
        out_shape=[
            jax.ShapeDtypeStruct((N, ND), jnp.float32),
            jax.ShapeDtypeStruct((N, HID), jnp.float32),
            jax.ShapeDtypeStruct((N, HID), jnp.float32),
        ],
    )(h_node, parts, parts, w1[:ND], w1[ND:], b1.reshape(1, HID), w2,
      b2.reshape(1, ND), ln_g.reshape(1, ND), ln_b.reshape(1, ND),
      w1s_next, w1d_next)


def _make_gather(e_total, chunk):
    epw = e_total // NW
    nchunks = epw // chunk
    assert epw % chunk == 0 and chunk % 8 == 0 and chunk <= 128
    assert nchunks % 2 == 1

    scratch = [
        pltpu.VMEM((nchunks, chunk), jnp.int32),
        pltpu.VMEM((nchunks, chunk), jnp.int32),
        pltpu.VMEM((chunk, HID), jnp.float32),
        pltpu.VMEM((chunk, HID), jnp.float32),
        pltpu.VMEM((chunk, HID), jnp.float32),
        pltpu.VMEM((chunk, HID), jnp.float32),
        pltpu.VMEM((chunk, HID), jnp.float32),
        pltpu.VMEM((chunk, HID), jnp.float32),
        pltpu.SemaphoreType.DMA,
        pltpu.SemaphoreType.DMA,
        pltpu.SemaphoreType.DMA,
        pltpu.SemaphoreType.DMA,
    ]

    def body(ps_hbm, pd_hbm, src_hbm, dst_hbm, out_hbm,
             si_all, di_all, rs0, rs1, rd0, rd1, ou0, ou1,
             gsem0, gsem1, wsem0, wsem1):
        wid = lax.axis_index("s") * NC + lax.axis_index("c")
        base = wid * epw
        rs = (rs0, rs1)
        rd = (rd0, rd1)
        ou = (ou0, ou1)
        gsem = (gsem0, gsem1)
        wsem = (wsem0, wsem1)

        pltpu.sync_copy(src_hbm.at[wid], si_all)
        pltpu.sync_copy(dst_hbm.at[wid], di_all)

        def issue(c, p):
            pltpu.async_copy(ps_hbm.at[si_all.at[c]], rs[p], gsem[p])
            pltpu.async_copy(pd_hbm.at[di_all.at[c]], rd[p], gsem[p])

        def process(c, p):
            pltpu.make_async_copy(
                ps_hbm.at[si_all.at[0]], rs[p], gsem[p]).wait()
            pltpu.make_async_copy(
                pd_hbm.at[di_all.at[0]], rd[p], gsem[p]).wait()

            @pl.when(c >= 2)
            def _():
                pltpu.make_async_copy(
                    ou[p], out_hbm.at[pl.ds(0, chunk)], wsem[p]).wait()

            def add_row(r, carry):
                for cix in range(HID // 16):
                    sl = pl.ds(cix * 16, 16)
                    ou[p][r, sl] = rs[p][r, sl] + rd[p][r, sl]
                return carry

            lax.fori_loop(0, chunk, add_row, 0)
            pltpu.async_copy(
                ou[p], out_hbm.at[pl.ds(base + c * chunk, chunk)], wsem[p])

        issue(0, 0)

        def loop_body(j2, carry):
            c0 = 2 * j2
            issue(c0 + 1, 1)
            process(c0, 0)
            issue(c0 + 2, 0)
            process(c0 + 1, 1)
            return carry

        lax.fori_loop(0, (nchunks - 1) // 2, loop_body, 0)
        process(nchunks - 1, 0)
        pltpu.make_async_copy(
            ou[0], out_hbm.at[pl.ds(0, chunk)], wsem[0]).wait()
        pltpu.make_async_copy(
            ou[1], out_hbm.at[pl.ds(0, chunk)], wsem[1]).wait()

    return pl.kernel(
        body,
        out_type=jax.ShapeDtypeStruct((e_total, HID), jnp.float32),
        mesh=_SC_MESH,
        scratch_types=scratch,
    )


_gather_add = _make_gather(E, CHUNK)

_SEGSUM_SCRATCH = [
    pltpu.VMEM((NCHUNKS, CHUNK), jnp.int32),
    pltpu.VMEM((CHUNK, ED), jnp.float32),
    pltpu.VMEM((CHUNK, ED), jnp.float32),
    pltpu.VMEM((CHUNK, ND), jnp.float32),
    pltpu.VMEM_SHARED((NACC, ND), jnp.float32),
    pltpu.SemaphoreType.DMA,
    pltpu.SemaphoreType.DMA,
    pltpu.SemaphoreType.DMA,
]


def _segment_sum_body(he_hbm, dst_hbm, out_hbm, di_all, r16a, r16b,
                      rows_v, acc_sh, lsem0, lsem1, ssem):
    c = lax.axis_index("c")
    s = lax.axis_index("s")
    wid = s * NC + c
    r16 = (r16a, r16b)
    lsem = (lsem0, lsem1)

    def zrow(r, carry):
        for cix in range(ND // 16):
            rows_v[r, pl.ds(cix * 16, 16)] = jnp.zeros((16,), jnp.float32)
        return carry

    lax.fori_loop(0, CHUNK, zrow, 0)
    for t in range(NACC // NS // CHUNK):
        pltpu.sync_copy(
            rows_v,
            acc_sh.at[pl.ds(s * (NACC // NS) + t * CHUNK, CHUNK)])
    plsc.subcore_barrier()

    pltpu.sync_copy(dst_hbm.at[wid], di_all)

    def issue(cc, p):
        pltpu.async_copy(he_hbm.at[wid, pl.ds(cc * CHUNK, CHUNK)],
                         r16[p], lsem[p])

    def process(cc, p):
        pltpu.make_async_copy(
            he_hbm.at[wid, pl.ds(0, CHUNK)], r16[p], lsem[p]).wait()

        @pl.when(cc >= 1)
        def _():
            pltpu.make_async_copy(
                rows_v, acc_sh.at[di_all.at[0]], ssem).wait()

        def fill(r, carry):
            rows_v[r, pl.ds(0, 16)] = r16[p][r, pl.ds(0, 16)]
            return carry

        lax.fori_loop(0, CHUNK, fill, 0)
        pltpu.async_copy(rows_v, acc_sh.at[di_all.at[cc]], ssem,
                         add=True)

    issue(0, 0)

    def loop_body(j2, carry):
        c0 = 2 * j2
        issue(c0 + 1, 1)
        process(c0, 0)
        issue(c0 + 2, 0)
        process(c0 + 1, 1)
        return carry

    lax.fori_loop(0, (NCHUNKS - 1) // 2, loop_body, 0)
    process(NCHUNKS - 1, 0)
    pltpu.make_async_copy(rows_v, acc_sh.at[di_all.at[0]], ssem).wait()
    plsc.subcore_barrier()

    @pl.when(s == 0)
    def _():
        pltpu.sync_copy(acc_sh, out_hbm.at[c])


_segment_sum = pl.kernel(
    _segment_sum_body,
    out_type=jax.ShapeDtypeStruct((NC, NACC, ND), jnp.float32),
    mesh=_SC_MESH,
    scratch_types=_SEGSUM_SCRATCH,
)


def kernel(h_node, edge_index, h_edge, edge_w1, edge_b1, edge_w2, edge_b2,
           edge_ln_g, edge_ln_b, node_w1, node_b1, node_w2, node_b2,
           node_ln_g, node_ln_b):
    src4 = edge_index[0].reshape(NW, NCHUNKS, CHUNK)
    dst4 = edge_index[1].reshape(NW, NCHUNKS, CHUNK)
    ps, pd = _proj(h_node, edge_w1[0, :ND], edge_w1[0, ND:2 * ND])
    for i in range(L):
        nxt = (i + 1) % L
        g = _gather_add(ps, pd, src4, dst4)
        h_edge = _edge_mlp(g, h_edge, edge_w1[i, 2 * ND:], edge_b1[i],
                           edge_w2[i], edge_b2[i], edge_ln_g[i], edge_ln_b[i])
        parts = _segment_sum(h_edge.reshape(NW, EPW, ED), dst4)
        h_node, ps, pd = _node_mlp(h_node, parts, node_w1[i], node_b1[i],
                                   node_w2[i], node_b2[i], node_ln_g[i],
                                   node_ln_b[i], edge_w1[nxt, :ND],
                                   edge_w1[nxt, ND:2 * ND])
    return (h_node, h_edge)


# speedup vs baseline: 1.8625x; 1.7245x over previous
import functools

import jax
import jax.numpy as jnp
from jax import lax
from jax.experimental import pallas as pl
from jax.experimental.pallas import tpu as pltpu
from jax.experimental.pallas import tpu_sc as plsc

N = 10000
E = 320000
ND = 128
ED = 16
HID = 128
L = 4

NC, NS = 2, 16
NW = NC * NS
EPW = E // NW
CHUNK = 80
NCHUNKS = EPW // CHUNK
NACC = 10240
ROWS_PT = NACC // NS

_SC_MESH = plsc.VectorSubcoreMesh(core_axis_name="c", subcore_axis_name="s",
                                  num_cores=NC, num_subcores=NS)


def _proj_body(hn_ref, ws_ref, wd_ref, ps_ref, pd_ref):
    hn = hn_ref[...]
    ps_ref[...] = jnp.dot(hn, ws_ref[...], preferred_element_type=jnp.float32, precision=lax.Precision.HIGHEST)
    pd_ref[...] = jnp.dot(hn, wd_ref[...], preferred_element_type=jnp.float32, precision=lax.Precision.HIGHEST)


def _proj(h_node, w1s, w1d):
    bn = 2000
    return pl.pallas_call(
        _proj_body,
        grid=(N // bn,),
        in_specs=[
            pl.BlockSpec((bn, ND), lambda i: (i, 0)),
            pl.BlockSpec((ND, HID), lambda i: (0, 0)),
            pl.BlockSpec((ND, HID), lambda i: (0, 0)),
        ],
        out_specs=[
            pl.BlockSpec((bn, HID), lambda i: (i, 0)),
            pl.BlockSpec((bn, HID), lambda i: (i, 0)),
        ],
        out_shape=[
            jax.ShapeDtypeStruct((N, HID), jnp.float32),
            jax.ShapeDtypeStruct((N, HID), jnp.float32),
        ],
    )(h_node, w1s, w1d)


def _edge_body(g_ref, he_ref, w1e_ref, b1_ref, w2_ref, b2_ref,
               lg_ref, lb_ref, out_ref):
    he = he_ref[...]
    pre = (g_ref[...]
           + jnp.dot(he, w1e_ref[...], preferred_element_type=jnp.float32)
           + b1_ref[...])
    h = jnp.maximum(pre, 0.0)
    e = jnp.dot(h, w2_ref[...], preferred_element_type=jnp.float32) + b2_ref[...]
    m = jnp.mean(e, axis=-1, keepdims=True)
    v = jnp.mean((e - m) ** 2, axis=-1, keepdims=True)
    e = (e - m) * lax.rsqrt(v + 1e-5) * lg_ref[...] + lb_ref[...]
    out_ref[...] = he + e


def _edge_mlp(g, h_edge, w1e, b1, w2, b2, ln_g, ln_b):
    e_len = g.shape[0]
    be = 8000
    return pl.pallas_call(
        _edge_body,
        grid=(e_len // be,),
        in_specs=[
            pl.BlockSpec((be, HID), lambda i: (i, 0)),
            pl.BlockSpec((be, ED), lambda i: (i, 0)),
            pl.BlockSpec((ED, HID), lambda i: (0, 0)),
            pl.BlockSpec((1, HID), lambda i: (0, 0)),
            pl.BlockSpec((HID, ED), lambda i: (0, 0)),
            pl.BlockSpec((1, ED), lambda i: (0, 0)),
            pl.BlockSpec((1, ED), lambda i: (0, 0)),
            pl.BlockSpec((1, ED), lambda i: (0, 0)),
        ],
        out_specs=pl.BlockSpec((be, ED), lambda i: (i, 0)),
        out_shape=jax.ShapeDtypeStruct((e_len, ED), jnp.float32),
    )(g, h_edge, w1e, b1.reshape(1, HID), w2, b2.reshape(1, ED),
      ln_g.reshape(1, ED), ln_b.reshape(1, ED))


def _node_body(hn_ref, a0_ref, a1_ref, w1n_ref, w1a_ref, b1_ref, w2_ref,
               b2_ref, lg_ref, lb_ref, ws_ref, wd_ref,
               out_ref, ps_ref, pd_ref):
    hn = hn_ref[...]
    agg = a0_ref[0, :, :ED] + a1_ref[0, :, :ED]
    pre = (jnp.dot(hn, w1n_ref[...], preferred_element_type=jnp.float32, precision=lax.Precision.HIGHEST)
           + jnp.dot(agg, w1a_ref[...], preferred_element_type=jnp.float32, precision=lax.Precision.HIGHEST)
           + b1_ref[...])
    h = jnp.maximum(pre, 0.0)
    nv = jnp.dot(h, w2_ref[...], preferred_element_type=jnp.float32, precision=lax.Precision.HIGHEST) + b2_ref[...]
    m = jnp.mean(nv, axis=-1, keepdims=True)
    v = jnp.mean((nv - m) ** 2, axis=-1, keepdims=True)
    nv = (nv - m) * lax.rsqrt(v + 1e-5) * lg_ref[...] + lb_ref[...]
    hn_new = hn + nv
    out_ref[...] = hn_new
    ps_ref[...] = jnp.dot(hn_new, ws_ref[...], preferred_element_type=jnp.float32, precision=lax.Precision.HIGHEST)
    pd_ref[...] = jnp.dot(hn_new, wd_ref[...], preferred_element_type=jnp.float32, precision=lax.Precision.HIGHEST)


def _node_mlp(h_node, parts, w1, b1, w2, b2, ln_g, ln_b, w1s_next, w1d_next):
    bn = 2000
    return pl.pallas_call(
        _node_body,
        grid=(N // bn,),
        in_specs=[
            pl.BlockSpec((bn, ND), lambda i: (i, 0)),
            pl.BlockSpec((1, bn, ND), lambda i: (0, i, 0)),
            pl.BlockSpec((1, bn, ND), lambda i: (1, i, 0)),
            pl.BlockSpec((ND, HID), lambda i: (0, 0)),
            pl.BlockSpec((ED, HID), lambda i: (0, 0)),
            pl.BlockSpec((1, HID), lambda i: (0, 0)),
            pl.BlockSpec((HID, ND), lambda i: (0, 0)),
            pl.BlockSpec((1, ND), lambda i: (0, 0)),
            pl.BlockSpec((1, ND), lambda i: (0, 0)),
            pl.BlockSpec((1, ND), lambda i: (0, 0)),
            pl.BlockSpec((ND, HID), lambda i: (0, 0)),
            pl.BlockSpec((ND, HID), lambda i: (0, 0)),
        ],
        out_specs=[
            pl.BlockSpec((bn, ND), lambda i: (i, 0)),
            pl.BlockSpec((bn, HID), lambda i: (i, 0)),
            pl.BlockSpec((bn, HID), lambda i: (i, 0)),
        ],
        out_shape=[
            jax.ShapeDtypeStruct((N, ND), jnp.float32),
            jax.ShapeDtypeStruct((N, HID), jnp.float32),
            jax.ShapeDtypeStruct((N, HID), jnp.float32),
        ],
    )(h_node, parts, parts, w1[:ND], w1[ND:], b1.reshape(1, HID), w2,
      b2.reshape(1, ND), ln_g.reshape(1, ND), ln_b.reshape(1, ND),
      w1s_next, w1d_next)


def _make_gather(e_total, chunk):
    epw = e_total // NW
    nchunks = epw // chunk
    assert epw % chunk == 0 and chunk % 8 == 0 and chunk <= 128
    assert nchunks % 2 == 1

    scratch = [
        pltpu.VMEM((nchunks, chunk), jnp.int32),
        pltpu.VMEM((nchunks, chunk), jnp.int32),
        pltpu.VMEM((chunk, HID), jnp.float32),
        pltpu.VMEM((chunk, HID), jnp.float32),
        pltpu.VMEM((chunk, HID), jnp.float32),
        pltpu.VMEM((chunk, HID), jnp.float32),
        pltpu.VMEM((chunk, HID), jnp.float32),
        pltpu.VMEM((chunk, HID), jnp.float32),
        pltpu.SemaphoreType.DMA,
        pltpu.SemaphoreType.DMA,
        pltpu.SemaphoreType.DMA,
        pltpu.SemaphoreType.DMA,
    ]

    def body(ps_hbm, pd_hbm, src_hbm, dst_hbm, out_hbm,
             si_all, di_all, rs0, rs1, rd0, rd1, ou0, ou1,
             gsem0, gsem1, wsem0, wsem1):
        wid = lax.axis_index("s") * NC + lax.axis_index("c")
        base = wid * epw
        rs = (rs0, rs1)
        rd = (rd0, rd1)
        ou = (ou0, ou1)
        gsem = (gsem0, gsem1)
        wsem = (wsem0, wsem1)

        pltpu.sync_copy(src_hbm.at[wid], si_all)
        pltpu.sync_copy(dst_hbm.at[wid], di_all)

        def issue(c, p):
            pltpu.async_copy(ps_hbm.at[si_all.at[c]], rs[p], gsem[p])
            pltpu.async_copy(pd_hbm.at[di_all.at[c]], rd[p], gsem[p])

        def process(c, p):
            pltpu.make_async_copy(
                ps_hbm.at[si_all.at[0]], rs[p], gsem[p]).wait()
            pltpu.make_async_copy(
                pd_hbm.at[di_all.at[0]], rd[p], gsem[p]).wait()

            @pl.when(c >= 2)
            def _():
                pltpu.make_async_copy(
                    ou[p], out_hbm.at[pl.ds(0, chunk)], wsem[p]).wait()

            def add_row(r, carry):
                for cix in range(HID // 16):
                    sl = pl.ds(cix * 16, 16)
                    ou[p][r, sl] = rs[p][r, sl] + rd[p][r, sl]
                return carry

            lax.fori_loop(0, chunk, add_row, 0)
            pltpu.async_copy(
                ou[p], out_hbm.at[pl.ds(base + c * chunk, chunk)], wsem[p])

        issue(0, 0)

        def loop_body(j2, carry):
            c0 = 2 * j2
            issue(c0 + 1, 1)
            process(c0, 0)
            issue(c0 + 2, 0)
            process(c0 + 1, 1)
            return carry

        lax.fori_loop(0, (nchunks - 1) // 2, loop_body, 0)
        process(nchunks - 1, 0)
        pltpu.make_async_copy(
            ou[0], out_hbm.at[pl.ds(0, chunk)], wsem[0]).wait()
        pltpu.make_async_copy(
            ou[1], out_hbm.at[pl.ds(0, chunk)], wsem[1]).wait()

    return pl.kernel(
        body,
        out_type=jax.ShapeDtypeStruct((e_total, HID), jnp.float32),
        mesh=_SC_MESH,
        scratch_types=scratch,
    )


_gather_add = _make_gather(E, CHUNK)

_SEGSUM_SCRATCH = [
    pltpu.VMEM((NCHUNKS, CHUNK), jnp.int32),
    pltpu.VMEM((CHUNK, ED), jnp.float32),
    pltpu.VMEM((CHUNK, ED), jnp.float32),
    pltpu.VMEM((CHUNK, ND), jnp.float32),
    pltpu.VMEM_SHARED((NACC, ND), jnp.float32),
    pltpu.SemaphoreType.DMA,
    pltpu.SemaphoreType.DMA,
    pltpu.SemaphoreType.DMA,
]


def _segment_sum_body(he_hbm, dst_hbm, out_hbm, di_all, r16a, r16b,
                      rows_v, acc_sh, lsem0, lsem1, ssem):
    c = lax.axis_index("c")
    s = lax.axis_index("s")
    wid = s * NC + c
    r16 = (r16a, r16b)
    lsem = (lsem0, lsem1)

    def zrow(r, carry):
        for cix in range(ND // 16):
            rows_v[r, pl.ds(cix * 16, 16)] = jnp.zeros((16,), jnp.float32)
        return carry

    lax.fori_loop(0, CHUNK, zrow, 0)
    for t in range(NACC // NS // CHUNK):
        pltpu.sync_copy(
            rows_v,
            acc_sh.at[pl.ds(s * (NACC // NS) + t * CHUNK, CHUNK)])
    plsc.subcore_barrier()

    pltpu.sync_copy(dst_hbm.at[wid], di_all)

    def issue(cc, p):
        pltpu.async_copy(he_hbm.at[wid, pl.ds(cc * CHUNK, CHUNK)],
                         r16[p], lsem[p])

    def process(cc, p):
        pltpu.make_async_copy(
            he_hbm.at[wid, pl.ds(0, CHUNK)], r16[p], lsem[p]).wait()

        @pl.when(cc >= 1)
        def _():
            pltpu.make_async_copy(
                rows_v, acc_sh.at[di_all.at[0]], ssem).wait()

        def fill(r, carry):
            rows_v[r, pl.ds(0, 16)] = r16[p][r, pl.ds(0, 16)]
            return carry

        lax.fori_loop(0, CHUNK, fill, 0)
        pltpu.async_copy(rows_v, acc_sh.at[di_all.at[cc]], ssem,
                         add=True)

    issue(0, 0)

    def loop_body(j2, carry):
        c0 = 2 * j2
        issue(c0 + 1, 1)
        process(c0, 0)
        issue(c0 + 2, 0)
        process(c0 + 1, 1)
        return carry

    lax.fori_loop(0, (NCHUNKS - 1) // 2, loop_body, 0)
    process(NCHUNKS - 1, 0)
    pltpu.make_async_copy(rows_v, acc_sh.at[di_all.at[0]], ssem).wait()
    plsc.subcore_barrier()

    @pl.when(s == 0)
    def _():
        pltpu.sync_copy(acc_sh, out_hbm.at[c])


_segment_sum = pl.kernel(
    _segment_sum_body,
    out_type=jax.ShapeDtypeStruct((NC, NACC, ND), jnp.float32),
    mesh=_SC_MESH,
    scratch_types=_SEGSUM_SCRATCH,
)


def kernel(h_node, edge_index, h_edge, edge_w1, edge_b1, edge_w2, edge_b2,
           edge_ln_g, edge_ln_b, node_w1, node_b1, node_w2, node_b2,
           node_ln_g, node_ln_b):
    src4 = edge_index[0].reshape(NW, NCHUNKS, CHUNK)
    dst4 = edge_index[1].reshape(NW, NCHUNKS, CHUNK)
    ps, pd = _proj(h_node, edge_w1[0, :ND], edge_w1[0, ND:2 * ND])
    for i in range(L):
        nxt = (i + 1) % L
        g = _gather_add(ps, pd, src4, dst4)
        h_edge = _edge_mlp(g, h_edge, edge_w1[i, 2 * ND:], edge_b1[i],
                           edge_w2[i], edge_b2[i], edge_ln_g[i], edge_ln_b[i])
        parts = _segment_sum(h_edge.reshape(NW, EPW, ED), dst4)
        h_node, ps, pd = _node_mlp(h_node, parts, node_w1[i], node_b1[i],
                                   node_w2[i], node_b2[i], node_ln_g[i],
                                   node_ln_b[i], edge_w1[nxt, :ND],
                                   edge_w1[nxt, ND:2 * ND])
    return (h_node, h_edge)
